# baseline (speedup 1.0000x reference)
import functools

import jax
import jax.numpy as jnp
from jax import lax
from jax.experimental import pallas as pl
from jax.experimental.pallas import tpu as pltpu
from jax.experimental.pallas import tpu_sc as plsc

T = 2048
D = 1024
DFF = 2816
NE = 8
EPAD = 128

TM = 128
NB = 40
GPAD = NB * TM
TFF = 256
NF = DFF // TFF

NC = 2
NS = 16
NW = NC * NS


def _router_body(x_ref, w_ref, b_ref, e1_ref, e2_ref, w1_ref, w2_ref):
    x = x_ref[...]
    w = w_ref[...]
    logits = lax.dot_general(x, w, (((1,), (1,)), ((), ())),
                             preferred_element_type=jnp.float32)
    logits = logits + b_ref[...]
    eidx = lax.broadcasted_iota(jnp.int32, (T, EPAD), 1)
    m1 = jnp.max(logits, axis=1, keepdims=True)
    e1 = jnp.min(jnp.where(logits >= m1, eidx, EPAD), axis=1, keepdims=True)
    l2 = jnp.where(eidx == e1, -1e30, logits)
    m2 = jnp.max(l2, axis=1, keepdims=True)
    e2 = jnp.min(jnp.where(l2 >= m2, eidx, EPAD), axis=1, keepdims=True)
    e1_ref[...] = e1
    e2_ref[...] = e2
    w1_ref[...] = jax.nn.sigmoid(m1 - m2)
    w2_ref[...] = jax.nn.sigmoid(m2 - m1)


def _run_router(flat, switch_w, switch_b):
    wpad = jnp.zeros((EPAD, D), jnp.float32).at[:NE].set(switch_w)
    bpad = jnp.full((1, EPAD), -1e30, jnp.float32).at[0, :NE].set(switch_b)
    return pl.pallas_call(
        _router_body,
        out_shape=(
            jax.ShapeDtypeStruct((T, 1), jnp.int32),
            jax.ShapeDtypeStruct((T, 1), jnp.int32),
            jax.ShapeDtypeStruct((T, 1), jnp.float32),
            jax.ShapeDtypeStruct((T, 1), jnp.float32),
        ),
    )(flat, wpad, bpad)


_GCH = 32
_GROWS = GPAD // NW


def _sc_mesh():
    return plsc.VectorSubcoreMesh(core_axis_name="c", subcore_axis_name="s",
                                  num_cores=NC, num_subcores=NS)


@functools.cache
def _make_sc_gather():
    @functools.partial(
        pl.kernel,
        out_type=jax.ShapeDtypeStruct((GPAD, D // 2), jnp.int32),
        mesh=_sc_mesh(),
        scratch_types=[
            pltpu.VMEM((_GCH,), jnp.int32),
            pltpu.VMEM((_GCH, D // 2), jnp.int32),
            pltpu.SemaphoreType.DMA,
        ],
    )
    def _sc_gather(x_hbm, idx_hbm, out_hbm, idx_v, rows_v, sem):
        c = lax.axis_index("c")
        s = lax.axis_index("s")
        base = (c * NS + s) * _GROWS

        def chunk(i, carry):
            off = base + i * _GCH
            pltpu.sync_copy(idx_hbm.at[pl.ds(off, _GCH)], idx_v)
            pltpu.async_copy(x_hbm.at[idx_v], rows_v, sem).wait()
            pltpu.sync_copy(rows_v, out_hbm.at[pl.ds(off, _GCH)])
            return carry

        lax.fori_loop(0, _GROWS // _GCH, chunk, 0)

    return _sc_gather


def _ffn_body(blk_ref, val_ref, x_ref, g_ref, u_ref, d_ref, cw_ref, y_ref,
              gwc_ref, uwc_ref, dwc_ref):
    f = pl.program_id(0)
    b = pl.program_id(1)

    changed = jnp.logical_or(b == 0,
                             blk_ref[b] != blk_ref[jnp.maximum(b - 1, 0)])

    @pl.when(changed)
    def _():
        gwc_ref[...] = g_ref[0].astype(jnp.bfloat16)
        uwc_ref[...] = u_ref[0].astype(jnp.bfloat16)
        dwc_ref[...] = d_ref[0].astype(jnp.bfloat16)

    @pl.when(val_ref[b] != 0)
    def _():
        rows = pl.ds(b * TM, TM)
        xb = x_ref[rows, :]
        g = lax.dot_general(xb, gwc_ref[...], (((1,), (1,)), ((), ())),
                            preferred_element_type=jnp.float32)
        u = lax.dot_general(xb, uwc_ref[...], (((1,), (1,)), ((), ())),
                            preferred_element_type=jnp.float32)
        p = (g * jax.nn.sigmoid(g) * u).astype(jnp.bfloat16)
        contrib = lax.dot_general(p, dwc_ref[...], (((1,), (1,)), ((), ())),
                                  preferred_element_type=jnp.float32)

        @pl.when(f == 0)
        def _():
            y_ref[rows, :] = contrib

        @pl.when(f != 0)
        def _():
            y_ref[rows, :] += contrib

        @pl.when(f == NF - 1)
        def _():
            y_ref[rows, :] *= cw_ref[...]


def _run_ffn(sorted_x, gate_w, up_w, down_w, cw_col, blk_e, blk_valid):
    grid_spec = pltpu.PrefetchScalarGridSpec(
        num_scalar_prefetch=2,
        grid=(NF, NB),
        in_specs=[
            pl.BlockSpec((GPAD, D), lambda f, b, blk, val: (0, 0)),
            pl.BlockSpec((1, TFF, D), lambda f, b, blk, val: (blk[b], f, 0)),
            pl.BlockSpec((1, TFF, D), lambda f, b, blk, val: (blk[b], f, 0)),
            pl.BlockSpec((1, D, TFF), lambda f, b, blk, val: (blk[b], 0, f)),
            pl.BlockSpec((TM, 1), lambda f, b, blk, val: (b, 0)),
        ],
        out_specs=pl.BlockSpec((GPAD, D), lambda f, b, blk, val: (0, 0)),
        scratch_shapes=[
            pltpu.VMEM((TFF, D), jnp.bfloat16),
            pltpu.VMEM((TFF, D), jnp.bfloat16),
            pltpu.VMEM((D, TFF), jnp.bfloat16),
        ],
    )
    return pl.pallas_call(
        _ffn_body,
        grid_spec=grid_spec,
        out_shape=jax.ShapeDtypeStruct((GPAD, D), jnp.float32),
        compiler_params=pltpu.CompilerParams(
            dimension_semantics=("arbitrary", "arbitrary")),
    )(blk_e, blk_valid, sorted_x, gate_w, up_w, down_w, cw_col)


_CCH = 32
_CTOK = T // NW


@functools.cache
def _make_sc_combine():
    @functools.partial(
        pl.kernel,
        out_type=jax.ShapeDtypeStruct((T, D), jnp.float32),
        mesh=_sc_mesh(),
        scratch_types=[
            pltpu.VMEM((_CCH,), jnp.int32),
            pltpu.VMEM((_CCH, D), jnp.float32),
            pltpu.VMEM((_CCH, D), jnp.float32),
            pltpu.SemaphoreType.DMA,
        ],
    )
    def _sc_combine(y_hbm, d1_hbm, d2_hbm, out_hbm, idx_v, rows_v, acc_v,
                    sem):
        c = lax.axis_index("c")
        s = lax.axis_index("s")
        goff = (c * NS + s) * _CTOK

        def chunk(i, carry):
            og = goff + i * _CCH
            pltpu.sync_copy(d1_hbm.at[pl.ds(og, _CCH)], idx_v)
            pltpu.async_copy(y_hbm.at[idx_v], acc_v, sem).wait()
            pltpu.sync_copy(d2_hbm.at[pl.ds(og, _CCH)], idx_v)
            pltpu.async_copy(y_hbm.at[idx_v], rows_v, sem).wait()

            def row_add(r, carry2):
                for k in range(D // 16):
                    sl = pl.ds(16 * k, 16)
                    acc_v[r, sl] += rows_v[r, sl]
                return carry2

            lax.fori_loop(0, _CCH, row_add, 0)
            pltpu.sync_copy(acc_v, out_hbm.at[pl.ds(og, _CCH)])
            return carry

        lax.fori_loop(0, _CTOK // _CCH, chunk, 0)

    return _sc_combine


def _sc_gather(flat, src_tok):
    return _make_sc_gather()(flat, src_tok)


def _sc_combine(y, d1, d2):
    return _make_sc_combine()(y, d1, d2)


def kernel(x, switch_w, switch_b, gate_w, up_w, down_w):
    bsz, n, d = x.shape
    flat = x.reshape(-1, d)

    e1, e2, w1, w2 = _run_router(flat, switch_w, switch_b)
    e1 = e1[:, 0]
    e2 = e2[:, 0]

    a = jnp.concatenate([e1, e2])
    oh = jax.nn.one_hot(a, NE, dtype=jnp.int32)
    ranks = jnp.cumsum(oh, axis=0) - oh
    rank = jnp.take_along_axis(ranks, a[:, None], axis=1)[:, 0]
    counts = jnp.sum(oh, axis=0)
    padded = ((counts + TM - 1) // TM) * TM
    cum_end = jnp.cumsum(padded)
    pad_off = cum_end - padded
    dest = pad_off[a] + rank
    used = cum_end[NE - 1]

    block_starts = jnp.arange(NB, dtype=jnp.int32) * TM
    blk_e = jnp.minimum(
        jnp.searchsorted(cum_end, block_starts, side="right"),
        NE - 1).astype(jnp.int32)
    valid = (block_starts < used)
    last_e = blk_e[jnp.maximum(used // TM - 1, 0)]
    blk_e = jnp.where(valid, blk_e, last_e)
    blk_valid = valid.astype(jnp.int32)

    tok = jnp.concatenate([jnp.arange(T, dtype=jnp.int32)] * 2)
    src_tok = jnp.zeros((GPAD,), jnp.int32).at[dest].set(tok)
    cw_assign = jnp.concatenate([w1[:, 0], w2[:, 0]])
    cw_col = jnp.zeros((GPAD, 1), jnp.float32).at[dest, 0].set(cw_assign)

    flat_pk = lax.bitcast_convert_type(
        flat.astype(jnp.bfloat16).reshape(T, D // 2, 2), jnp.int32)
    sorted_pk = _sc_gather(flat_pk, src_tok)
    sorted_x = lax.bitcast_convert_type(
        sorted_pk, jnp.bfloat16).reshape(GPAD, D)
    y = _run_ffn(sorted_x, gate_w, up_w, down_w, cw_col, blk_e, blk_valid)
    out = _sc_combine(y, dest[:T].astype(jnp.int32),
                      dest[T:].astype(jnp.int32))
    return out.reshape(bsz, n, d)


# speedup vs baseline: 1.2547x; 1.2547x over previous
import functools

import jax
import jax.numpy as jnp
from jax import lax
from jax.experimental import pallas as pl
from jax.experimental.pallas import tpu as pltpu
from jax.experimental.pallas import tpu_sc as plsc

T = 2048
D = 1024
DFF = 2816
NE = 8
EPAD = 128

TM = 128
NB = 40
GPAD = NB * TM
TFF = 256
NF = DFF // TFF

NC = 2
NS = 16
NW = NC * NS


def _router_body(x_ref, w_ref, b_ref, e1_ref, e2_ref, w1_ref, w2_ref):
    x = x_ref[...]
    w = w_ref[...]
    logits = lax.dot_general(x, w, (((1,), (1,)), ((), ())),
                             preferred_element_type=jnp.float32)
    logits = logits + b_ref[...]
    eidx = lax.broadcasted_iota(jnp.int32, (T, EPAD), 1)
    m1 = jnp.max(logits, axis=1, keepdims=True)
    e1 = jnp.min(jnp.where(logits >= m1, eidx, EPAD), axis=1, keepdims=True)
    l2 = jnp.where(eidx == e1, -1e30, logits)
    m2 = jnp.max(l2, axis=1, keepdims=True)
    e2 = jnp.min(jnp.where(l2 >= m2, eidx, EPAD), axis=1, keepdims=True)
    e1_ref[...] = e1
    e2_ref[...] = e2
    w1_ref[...] = jax.nn.sigmoid(m1 - m2)
    w2_ref[...] = jax.nn.sigmoid(m2 - m1)


def _run_router(flat, switch_w, switch_b):
    wpad = jnp.zeros((EPAD, D), jnp.float32).at[:NE].set(switch_w)
    bpad = jnp.full((1, EPAD), -1e30, jnp.float32).at[0, :NE].set(switch_b)
    return pl.pallas_call(
        _router_body,
        out_shape=(
            jax.ShapeDtypeStruct((T, 1), jnp.int32),
            jax.ShapeDtypeStruct((T, 1), jnp.int32),
            jax.ShapeDtypeStruct((T, 1), jnp.float32),
            jax.ShapeDtypeStruct((T, 1), jnp.float32),
        ),
    )(flat, wpad, bpad)


_GCH = 32
_GROWS = GPAD // NW


def _sc_mesh():
    return plsc.VectorSubcoreMesh(core_axis_name="c", subcore_axis_name="s",
                                  num_cores=NC, num_subcores=NS)


@functools.cache
def _make_sc_gather():
    @functools.partial(
        pl.kernel,
        out_type=jax.ShapeDtypeStruct((GPAD, D), jnp.float32),
        mesh=_sc_mesh(),
        scratch_types=[
            pltpu.VMEM((_GCH,), jnp.int32),
            pltpu.VMEM((_GCH, D), jnp.float32),
            pltpu.SemaphoreType.DMA,
        ],
    )
    def _sc_gather(x_hbm, idx_hbm, out_hbm, idx_v, rows_v, sem):
        c = lax.axis_index("c")
        s = lax.axis_index("s")
        base = (c * NS + s) * _GROWS

        def chunk(i, carry):
            off = base + i * _GCH
            pltpu.sync_copy(idx_hbm.at[pl.ds(off, _GCH)], idx_v)
            pltpu.async_copy(x_hbm.at[idx_v], rows_v, sem).wait()
            pltpu.sync_copy(rows_v, out_hbm.at[pl.ds(off, _GCH)])
            return carry

        lax.fori_loop(0, _GROWS // _GCH, chunk, 0)

    return _sc_gather


def _ffn_body(blk_ref, val_ref, x_ref, g_ref, u_ref, d_ref, cw_ref, y_ref):
    f = pl.program_id(0)
    b = pl.program_id(1)

    @pl.when(val_ref[b] != 0)
    def _():
        rows = pl.ds(b * TM, TM)
        xb = x_ref[rows, :].astype(jnp.bfloat16)
        g = lax.dot_general(xb, g_ref[0].astype(jnp.bfloat16),
                            (((1,), (1,)), ((), ())),
                            preferred_element_type=jnp.float32)
        u = lax.dot_general(xb, u_ref[0].astype(jnp.bfloat16),
                            (((1,), (1,)), ((), ())),
                            preferred_element_type=jnp.float32)
        p = (g * jax.nn.sigmoid(g) * u).astype(jnp.bfloat16)
        contrib = lax.dot_general(p, d_ref[0].astype(jnp.bfloat16),
                                  (((1,), (1,)), ((), ())),
                                  preferred_element_type=jnp.float32)

        @pl.when(f == 0)
        def _():
            y_ref[rows, :] = contrib

        @pl.when(f != 0)
        def _():
            y_ref[rows, :] += contrib

        @pl.when(f == NF - 1)
        def _():
            y_ref[rows, :] *= cw_ref[...]


def _run_ffn(sorted_x, gate_w, up_w, down_w, cw_col, blk_e, blk_valid):
    grid_spec = pltpu.PrefetchScalarGridSpec(
        num_scalar_prefetch=2,
        grid=(NF, NB),
        in_specs=[
            pl.BlockSpec((GPAD, D), lambda f, b, blk, val: (0, 0)),
            pl.BlockSpec((1, TFF, D), lambda f, b, blk, val: (blk[b], f, 0)),
            pl.BlockSpec((1, TFF, D), lambda f, b, blk, val: (blk[b], f, 0)),
            pl.BlockSpec((1, D, TFF), lambda f, b, blk, val: (blk[b], 0, f)),
            pl.BlockSpec((TM, 1), lambda f, b, blk, val: (b, 0)),
        ],
        out_specs=pl.BlockSpec((GPAD, D), lambda f, b, blk, val: (0, 0)),
    )
    return pl.pallas_call(
        _ffn_body,
        grid_spec=grid_spec,
        out_shape=jax.ShapeDtypeStruct((GPAD, D), jnp.float32),
        compiler_params=pltpu.CompilerParams(
            dimension_semantics=("arbitrary", "arbitrary")),
    )(blk_e, blk_valid, sorted_x, gate_w, up_w, down_w, cw_col)


_CCH = 32
_CTOK = T // NW


@functools.cache
def _make_sc_combine():
    @functools.partial(
        pl.kernel,
        out_type=jax.ShapeDtypeStruct((T, D), jnp.float32),
        mesh=_sc_mesh(),
        scratch_types=[
            pltpu.VMEM((2 * _CCH,), jnp.int32),
            pltpu.VMEM((2 * _CCH, D), jnp.float32),
            pltpu.VMEM((_CCH, D), jnp.float32),
            pltpu.SemaphoreType.DMA,
        ],
    )
    def _sc_combine(y_hbm, dint_hbm, out_hbm, idx_v, rows_v, acc_v, sem):
        c = lax.axis_index("c")
        s = lax.axis_index("s")
        goff = (c * NS + s) * _CTOK

        def chunk(i, carry):
            og = goff + i * _CCH
            pltpu.sync_copy(dint_hbm.at[pl.ds(2 * og, 2 * _CCH)], idx_v)
            pltpu.async_copy(y_hbm.at[idx_v], rows_v, sem).wait()

            def row_add(r, carry2):
                for k in range(D // 16):
                    sl = pl.ds(16 * k, 16)
                    acc_v[r, sl] = rows_v[2 * r, sl] + rows_v[2 * r + 1, sl]
                return carry2

            lax.fori_loop(0, _CCH, row_add, 0)
            pltpu.sync_copy(acc_v, out_hbm.at[pl.ds(og, _CCH)])
            return carry

        lax.fori_loop(0, _CTOK // _CCH, chunk, 0)

    return _sc_combine


def _sc_gather(flat, src_tok):
    return _make_sc_gather()(flat, src_tok)


def _sc_combine(y, dint):
    return _make_sc_combine()(y, dint)


def kernel(x, switch_w, switch_b, gate_w, up_w, down_w):
    bsz, n, d = x.shape
    flat = x.reshape(-1, d)

    e1, e2, w1, w2 = _run_router(flat, switch_w, switch_b)
    e1 = e1[:, 0]
    e2 = e2[:, 0]

    a = jnp.concatenate([e1, e2])
    oh = jax.nn.one_hot(a, NE, dtype=jnp.int32)
    ranks = jnp.cumsum(oh, axis=0) - oh
    rank = jnp.take_along_axis(ranks, a[:, None], axis=1)[:, 0]
    counts = jnp.sum(oh, axis=0)
    padded = ((counts + TM - 1) // TM) * TM
    cum_end = jnp.cumsum(padded)
    pad_off = cum_end - padded
    dest = pad_off[a] + rank
    used = cum_end[NE - 1]

    block_starts = jnp.arange(NB, dtype=jnp.int32) * TM
    blk_e = jnp.minimum(
        jnp.searchsorted(cum_end, block_starts, side="right"),
        NE - 1).astype(jnp.int32)
    valid = (block_starts < used)
    last_e = blk_e[jnp.maximum(used // TM - 1, 0)]
    blk_e = jnp.where(valid, blk_e, last_e)
    blk_valid = valid.astype(jnp.int32)

    tok = jnp.concatenate([jnp.arange(T, dtype=jnp.int32)] * 2)
    src_tok = jnp.zeros((GPAD,), jnp.int32).at[dest].set(tok)
    cw_assign = jnp.concatenate([w1[:, 0], w2[:, 0]])
    cw_col = jnp.zeros((GPAD, 1), jnp.float32).at[dest, 0].set(cw_assign)

    sorted_x = _sc_gather(flat, src_tok)
    y = _run_ffn(sorted_x, gate_w, up_w, down_w, cw_col, blk_e, blk_valid)
    dint = jnp.stack([dest[:T], dest[T:]], axis=1).reshape(-1)
    out = _sc_combine(y, dint.astype(jnp.int32))
    return out.reshape(bsz, n, d)


# speedup vs baseline: 1.2804x; 1.0205x over previous
import functools

import jax
import jax.numpy as jnp
from jax import lax
from jax.experimental import pallas as pl
from jax.experimental.pallas import tpu as pltpu
from jax.experimental.pallas import tpu_sc as plsc

T = 2048
D = 1024
DFF = 2816
NE = 8
EPAD = 128

TM = 128
NB = 40
GPAD = NB * TM
TFF = 256
NF = DFF // TFF

NC = 2
NS = 16
NW = NC * NS


def _router_body(x_ref, w_ref, b_ref, e1_ref, e2_ref, w1_ref, w2_ref):
    x = x_ref[...]
    w = w_ref[...]
    logits = lax.dot_general(x, w, (((1,), (1,)), ((), ())),
                             preferred_element_type=jnp.float32)
    logits = logits + b_ref[...]
    eidx = lax.broadcasted_iota(jnp.int32, (T, EPAD), 1)
    m1 = jnp.max(logits, axis=1, keepdims=True)
    e1 = jnp.min(jnp.where(logits >= m1, eidx, EPAD), axis=1, keepdims=True)
    l2 = jnp.where(eidx == e1, -1e30, logits)
    m2 = jnp.max(l2, axis=1, keepdims=True)
    e2 = jnp.min(jnp.where(l2 >= m2, eidx, EPAD), axis=1, keepdims=True)
    e1_ref[...] = e1
    e2_ref[...] = e2
    w1_ref[...] = jax.nn.sigmoid(m1 - m2)
    w2_ref[...] = jax.nn.sigmoid(m2 - m1)


def _run_router(flat, switch_w, switch_b):
    wpad = jnp.zeros((EPAD, D), jnp.float32).at[:NE].set(switch_w)
    bpad = jnp.full((1, EPAD), -1e30, jnp.float32).at[0, :NE].set(switch_b)
    return pl.pallas_call(
        _router_body,
        out_shape=(
            jax.ShapeDtypeStruct((T, 1), jnp.int32),
            jax.ShapeDtypeStruct((T, 1), jnp.int32),
            jax.ShapeDtypeStruct((T, 1), jnp.float32),
            jax.ShapeDtypeStruct((T, 1), jnp.float32),
        ),
    )(flat, wpad, bpad)


_GCH = 32
_GROWS = GPAD // NW


def _sc_mesh():
    return plsc.VectorSubcoreMesh(core_axis_name="c", subcore_axis_name="s",
                                  num_cores=NC, num_subcores=NS)


@functools.cache
def _make_sc_gather():
    nch = _GROWS // _GCH

    @functools.partial(
        pl.kernel,
        out_type=jax.ShapeDtypeStruct((GPAD, D), jnp.float32),
        mesh=_sc_mesh(),
        scratch_types=[
            pltpu.VMEM((_GROWS,), jnp.int32),
            pltpu.VMEM((2, _GCH, D), jnp.float32),
            pltpu.SemaphoreType.DMA,
            pltpu.SemaphoreType.DMA,
        ],
    )
    def _sc_gather(x_hbm, idx_hbm, out_hbm, idx_v, bufs_v, sem0, sem1):
        c = lax.axis_index("c")
        s = lax.axis_index("s")
        base = (c * NS + s) * _GROWS
        sems = (sem0, sem1)
        pltpu.sync_copy(idx_hbm.at[pl.ds(base, _GROWS)], idx_v)

        handles = [None, None]
        handles[0] = pltpu.async_copy(
            x_hbm.at[idx_v.at[pl.ds(0, _GCH)]], bufs_v.at[0], sems[0])
        for i in range(nch):
            if i + 1 < nch:
                handles[(i + 1) % 2] = pltpu.async_copy(
                    x_hbm.at[idx_v.at[pl.ds((i + 1) * _GCH, _GCH)]],
                    bufs_v.at[(i + 1) % 2], sems[(i + 1) % 2])
            handles[i % 2].wait()
            pltpu.sync_copy(bufs_v.at[i % 2],
                            out_hbm.at[pl.ds(base + i * _GCH, _GCH)])

    return _sc_gather


def _ffn_body(blk_ref, val_ref, x_ref, g_ref, u_ref, d_ref, cw_ref, y_ref):
    f = pl.program_id(0)
    b = pl.program_id(1)

    @pl.when(val_ref[b] != 0)
    def _():
        rows = pl.ds(b * TM, TM)
        xb = x_ref[rows, :].astype(jnp.bfloat16)
        g = lax.dot_general(xb, g_ref[0].astype(jnp.bfloat16),
                            (((1,), (1,)), ((), ())),
                            preferred_element_type=jnp.float32)
        u = lax.dot_general(xb, u_ref[0].astype(jnp.bfloat16),
                            (((1,), (1,)), ((), ())),
                            preferred_element_type=jnp.float32)
        p = (g * jax.nn.sigmoid(g) * u).astype(jnp.bfloat16)
        contrib = lax.dot_general(p, d_ref[0].astype(jnp.bfloat16),
                                  (((1,), (1,)), ((), ())),
                                  preferred_element_type=jnp.float32)

        @pl.when(f == 0)
        def _():
            y_ref[rows, :] = contrib

        @pl.when(f != 0)
        def _():
            y_ref[rows, :] += contrib

        @pl.when(f == NF - 1)
        def _():
            y_ref[rows, :] *= cw_ref[...]


def _run_ffn(sorted_x, gate_w, up_w, down_w, cw_col, blk_e, blk_valid):
    grid_spec = pltpu.PrefetchScalarGridSpec(
        num_scalar_prefetch=2,
        grid=(NF, NB),
        in_specs=[
            pl.BlockSpec((GPAD, D), lambda f, b, blk, val: (0, 0)),
            pl.BlockSpec((1, TFF, D), lambda f, b, blk, val: (blk[b], f, 0)),
            pl.BlockSpec((1, TFF, D), lambda f, b, blk, val: (blk[b], f, 0)),
            pl.BlockSpec((1, D, TFF), lambda f, b, blk, val: (blk[b], 0, f)),
            pl.BlockSpec((TM, 1), lambda f, b, blk, val: (b, 0)),
        ],
        out_specs=pl.BlockSpec((GPAD, D), lambda f, b, blk, val: (0, 0)),
    )
    return pl.pallas_call(
        _ffn_body,
        grid_spec=grid_spec,
        out_shape=jax.ShapeDtypeStruct((GPAD, D), jnp.float32),
        compiler_params=pltpu.CompilerParams(
            dimension_semantics=("arbitrary", "arbitrary")),
    )(blk_e, blk_valid, sorted_x, gate_w, up_w, down_w, cw_col)


_CCH = 32
_CTOK = T // NW


@functools.cache
def _make_sc_combine():
    @functools.partial(
        pl.kernel,
        out_type=jax.ShapeDtypeStruct((T, D), jnp.float32),
        mesh=_sc_mesh(),
        scratch_types=[
            pltpu.VMEM((_CCH,), jnp.int32),
            pltpu.VMEM((_CCH, D), jnp.float32),
            pltpu.VMEM((_CCH, D), jnp.float32),
            pltpu.SemaphoreType.DMA,
        ],
    )
    def _sc_combine(y_hbm, d1_hbm, d2_hbm, out_hbm, idx_v, rows_v, acc_v,
                    sem):
        c = lax.axis_index("c")
        s = lax.axis_index("s")
        goff = (c * NS + s) * _CTOK

        def chunk(i, carry):
            og = goff + i * _CCH
            pltpu.sync_copy(d1_hbm.at[pl.ds(og, _CCH)], idx_v)
            pltpu.async_copy(y_hbm.at[idx_v], acc_v, sem).wait()
            pltpu.sync_copy(d2_hbm.at[pl.ds(og, _CCH)], idx_v)
            pltpu.async_copy(y_hbm.at[idx_v], rows_v, sem).wait()

            def row_add(r, carry2):
                for k in range(D // 16):
                    sl = pl.ds(16 * k, 16)
                    acc_v[r, sl] += rows_v[r, sl]
                return carry2

            lax.fori_loop(0, _CCH, row_add, 0)
            pltpu.sync_copy(acc_v, out_hbm.at[pl.ds(og, _CCH)])
            return carry

        lax.fori_loop(0, _CTOK // _CCH, chunk, 0)

    return _sc_combine


def _sc_gather(flat, src_tok):
    return _make_sc_gather()(flat, src_tok)


def _sc_combine(y, d1, d2):
    return _make_sc_combine()(y, d1, d2)


def kernel(x, switch_w, switch_b, gate_w, up_w, down_w):
    bsz, n, d = x.shape
    flat = x.reshape(-1, d)

    e1, e2, w1, w2 = _run_router(flat, switch_w, switch_b)
    e1 = e1[:, 0]
    e2 = e2[:, 0]

    a = jnp.concatenate([e1, e2])
    oh = jax.nn.one_hot(a, NE, dtype=jnp.int32)
    ranks = jnp.cumsum(oh, axis=0) - oh
    rank = jnp.take_along_axis(ranks, a[:, None], axis=1)[:, 0]
    counts = jnp.sum(oh, axis=0)
    padded = ((counts + TM - 1) // TM) * TM
    cum_end = jnp.cumsum(padded)
    pad_off = cum_end - padded
    dest = pad_off[a] + rank
    used = cum_end[NE - 1]

    block_starts = jnp.arange(NB, dtype=jnp.int32) * TM
    blk_e = jnp.minimum(
        jnp.searchsorted(cum_end, block_starts, side="right"),
        NE - 1).astype(jnp.int32)
    valid = (block_starts < used)
    last_e = blk_e[jnp.maximum(used // TM - 1, 0)]
    blk_e = jnp.where(valid, blk_e, last_e)
    blk_valid = valid.astype(jnp.int32)

    tok = jnp.concatenate([jnp.arange(T, dtype=jnp.int32)] * 2)
    src_tok = jnp.zeros((GPAD,), jnp.int32).at[dest].set(tok)
    cw_assign = jnp.concatenate([w1[:, 0], w2[:, 0]])
    cw_col = jnp.zeros((GPAD, 1), jnp.float32).at[dest, 0].set(cw_assign)

    sorted_x = _sc_gather(flat, src_tok)
    y = _run_ffn(sorted_x, gate_w, up_w, down_w, cw_col, blk_e, blk_valid)
    out = _sc_combine(y, dest[:T].astype(jnp.int32),
                      dest[T:].astype(jnp.int32))
    return out.reshape(bsz, n, d)


# speedup vs baseline: 1.5814x; 1.2351x over previous
import functools

import jax
import jax.numpy as jnp
from jax import lax
from jax.experimental import pallas as pl
from jax.experimental.pallas import tpu as pltpu
from jax.experimental.pallas import tpu_sc as plsc

T = 2048
D = 1024
DFF = 2816
NE = 8
EPAD = 128

TM = 128
NB = 40
GPAD = NB * TM
TFF = 256
NF = DFF // TFF

NC = 2
NS = 16
NW = NC * NS


def _router_body(x_ref, w_ref, b_ref, e1_ref, e2_ref, w1_ref, w2_ref):
    x = x_ref[...]
    w = w_ref[...]
    logits = lax.dot_general(x, w, (((1,), (1,)), ((), ())),
                             preferred_element_type=jnp.float32)
    logits = logits + b_ref[...]
    eidx = lax.broadcasted_iota(jnp.int32, (T, EPAD), 1)
    m1 = jnp.max(logits, axis=1, keepdims=True)
    e1 = jnp.min(jnp.where(logits >= m1, eidx, EPAD), axis=1, keepdims=True)
    l2 = jnp.where(eidx == e1, -1e30, logits)
    m2 = jnp.max(l2, axis=1, keepdims=True)
    e2 = jnp.min(jnp.where(l2 >= m2, eidx, EPAD), axis=1, keepdims=True)
    e1_ref[...] = e1
    e2_ref[...] = e2
    w1_ref[...] = jax.nn.sigmoid(m1 - m2)
    w2_ref[...] = jax.nn.sigmoid(m2 - m1)


def _run_router(flat, switch_w, switch_b):
    wpad = jnp.zeros((EPAD, D), jnp.float32).at[:NE].set(switch_w)
    bpad = jnp.full((1, EPAD), -1e30, jnp.float32).at[0, :NE].set(switch_b)
    return pl.pallas_call(
        _router_body,
        out_shape=(
            jax.ShapeDtypeStruct((T, 1), jnp.int32),
            jax.ShapeDtypeStruct((T, 1), jnp.int32),
            jax.ShapeDtypeStruct((T, 1), jnp.float32),
            jax.ShapeDtypeStruct((T, 1), jnp.float32),
        ),
    )(flat, wpad, bpad)


_GCH = 32
_GROWS = GPAD // NW


def _sc_mesh():
    return plsc.VectorSubcoreMesh(core_axis_name="c", subcore_axis_name="s",
                                  num_cores=NC, num_subcores=NS)


@functools.cache
def _make_sc_gather():
    nch = _GROWS // _GCH

    @functools.partial(
        pl.kernel,
        out_type=jax.ShapeDtypeStruct((GPAD, D), jnp.float32),
        mesh=_sc_mesh(),
        scratch_types=[
            pltpu.VMEM((_GROWS,), jnp.int32),
            pltpu.VMEM((2, _GCH, D), jnp.float32),
            pltpu.SemaphoreType.DMA,
            pltpu.SemaphoreType.DMA,
        ],
    )
    def _sc_gather(x_hbm, idx_hbm, out_hbm, idx_v, bufs_v, sem0, sem1):
        c = lax.axis_index("c")
        s = lax.axis_index("s")
        base = (c * NS + s) * _GROWS
        sems = (sem0, sem1)
        pltpu.sync_copy(idx_hbm.at[pl.ds(base, _GROWS)], idx_v)

        handles = [None, None]
        handles[0] = pltpu.async_copy(
            x_hbm.at[idx_v.at[pl.ds(0, _GCH)]], bufs_v.at[0], sems[0])
        for i in range(nch):
            if i + 1 < nch:
                handles[(i + 1) % 2] = pltpu.async_copy(
                    x_hbm.at[idx_v.at[pl.ds((i + 1) * _GCH, _GCH)]],
                    bufs_v.at[(i + 1) % 2], sems[(i + 1) % 2])
            handles[i % 2].wait()
            pltpu.sync_copy(bufs_v.at[i % 2],
                            out_hbm.at[pl.ds(base + i * _GCH, _GCH)])

    return _sc_gather


def _ffn_body(meta_ref, x_ref, gw_hbm, uw_hbm, dw_hbm, cw_ref, y_ref,
              gb0, gb1, ub0, ub1, db0, db1, g16, u16, d16,
              gs0, gs1, us0, us1, ds0, ds1):
    nruns = meta_ref[0]
    kmax = NF * nruns
    gbufs = (gb0, gb1)
    ubufs = (ub0, ub1)
    dbufs = (db0, db1)
    gsems = (gs0, gs1)
    usems = (us0, us1)
    dsems = (ds0, ds1)

    def issue(slot, e, f):
        foff = f * TFF
        pltpu.async_copy(gw_hbm.at[e, pl.ds(foff, TFF), :], gbufs[slot],
                         gsems[slot])
        pltpu.async_copy(uw_hbm.at[e, pl.ds(foff, TFF), :], ubufs[slot],
                         usems[slot])
        pltpu.async_copy(dw_hbm.at[e, :, pl.ds(foff, TFF)], dbufs[slot],
                         dsems[slot])

    def wait(slot):
        pltpu.make_async_copy(gw_hbm.at[0, pl.ds(0, TFF), :], gbufs[slot],
                              gsems[slot]).wait()
        pltpu.make_async_copy(uw_hbm.at[0, pl.ds(0, TFF), :], ubufs[slot],
                              usems[slot]).wait()
        pltpu.make_async_copy(dw_hbm.at[0, :, pl.ds(0, TFF)], dbufs[slot],
                              dsems[slot]).wait()

    issue(0, meta_ref[1], 0)

    def chunk(k, carry):
        f = k // nruns
        r = k - f * nruns
        kn = k + 1

        @pl.when(kn < kmax)
        def _():
            fn = kn // nruns
            rn = kn - fn * nruns
            en = meta_ref[1 + rn]

            @pl.when(kn % 2 == 0)
            def _():
                issue(0, en, fn)

            @pl.when(kn % 2 == 1)
            def _():
                issue(1, en, fn)

        @pl.when(k % 2 == 0)
        def _():
            wait(0)
            g16[...] = gb0[...].astype(jnp.bfloat16)
            u16[...] = ub0[...].astype(jnp.bfloat16)
            d16[...] = db0[...].astype(jnp.bfloat16)

        @pl.when(k % 2 == 1)
        def _():
            wait(1)
            g16[...] = gb1[...].astype(jnp.bfloat16)
            u16[...] = ub1[...].astype(jnp.bfloat16)
            d16[...] = db1[...].astype(jnp.bfloat16)

        b0 = meta_ref[1 + NE + r]
        nb = meta_ref[1 + 2 * NE + r]

        def block(j, carry2):
            rows = pl.ds((b0 + j) * TM, TM)
            xb = x_ref[rows, :]
            g = lax.dot_general(xb, g16[...], (((1,), (1,)), ((), ())),
                                preferred_element_type=jnp.float32)
            u = lax.dot_general(xb, u16[...], (((1,), (1,)), ((), ())),
                                preferred_element_type=jnp.float32)
            p = (g * jax.nn.sigmoid(g) * u).astype(jnp.bfloat16)
            contrib = lax.dot_general(p, d16[...], (((1,), (1,)), ((), ())),
                                      preferred_element_type=jnp.float32)

            @pl.when(f == 0)
            def _():
                y_ref[rows, :] = contrib

            @pl.when(f != 0)
            def _():
                y_ref[rows, :] += contrib

            @pl.when(f == NF - 1)
            def _():
                y_ref[rows, :] *= cw_ref[rows, :]

            return carry2

        lax.fori_loop(0, nb, block, 0)
        return carry

    lax.fori_loop(0, kmax, chunk, 0)


def _run_ffn(sorted_x16, gate_w, up_w, down_w, cw_col, meta):
    return pl.pallas_call(
        _ffn_body,
        in_specs=[
            pl.BlockSpec(memory_space=pltpu.MemorySpace.SMEM),
            pl.BlockSpec(memory_space=pltpu.MemorySpace.VMEM),
            pl.BlockSpec(memory_space=pl.ANY),
            pl.BlockSpec(memory_space=pl.ANY),
            pl.BlockSpec(memory_space=pl.ANY),
            pl.BlockSpec(memory_space=pltpu.MemorySpace.VMEM),
        ],
        out_specs=pl.BlockSpec(memory_space=pltpu.MemorySpace.VMEM),
        out_shape=jax.ShapeDtypeStruct((GPAD, D), jnp.float32),
        scratch_shapes=[
            pltpu.VMEM((TFF, D), jnp.float32),
            pltpu.VMEM((TFF, D), jnp.float32),
            pltpu.VMEM((TFF, D), jnp.float32),
            pltpu.VMEM((TFF, D), jnp.float32),
            pltpu.VMEM((D, TFF), jnp.float32),
            pltpu.VMEM((D, TFF), jnp.float32),
            pltpu.VMEM((TFF, D), jnp.bfloat16),
            pltpu.VMEM((TFF, D), jnp.bfloat16),
            pltpu.VMEM((D, TFF), jnp.bfloat16),
            pltpu.SemaphoreType.DMA,
            pltpu.SemaphoreType.DMA,
            pltpu.SemaphoreType.DMA,
            pltpu.SemaphoreType.DMA,
            pltpu.SemaphoreType.DMA,
            pltpu.SemaphoreType.DMA,
        ],
    )(meta, sorted_x16, gate_w, up_w, down_w, cw_col)


_CCH = 32
_CTOK = T // NW


@functools.cache
def _make_sc_combine():
    @functools.partial(
        pl.kernel,
        out_type=jax.ShapeDtypeStruct((T, D), jnp.float32),
        mesh=_sc_mesh(),
        scratch_types=[
            pltpu.VMEM((_CCH,), jnp.int32),
            pltpu.VMEM((_CCH, D), jnp.float32),
            pltpu.VMEM((_CCH, D), jnp.float32),
            pltpu.SemaphoreType.DMA,
        ],
    )
    def _sc_combine(y_hbm, d1_hbm, d2_hbm, out_hbm, idx_v, rows_v, acc_v,
                    sem):
        c = lax.axis_index("c")
        s = lax.axis_index("s")
        goff = (c * NS + s) * _CTOK

        def chunk(i, carry):
            og = goff + i * _CCH
            pltpu.sync_copy(d1_hbm.at[pl.ds(og, _CCH)], idx_v)
            pltpu.async_copy(y_hbm.at[idx_v], acc_v, sem).wait()
            pltpu.sync_copy(d2_hbm.at[pl.ds(og, _CCH)], idx_v)
            pltpu.async_copy(y_hbm.at[idx_v], rows_v, sem).wait()

            def row_add(r, carry2):
                for k in range(D // 16):
                    sl = pl.ds(16 * k, 16)
                    acc_v[r, sl] += rows_v[r, sl]
                return carry2

            lax.fori_loop(0, _CCH, row_add, 0)
            pltpu.sync_copy(acc_v, out_hbm.at[pl.ds(og, _CCH)])
            return carry

        lax.fori_loop(0, _CTOK // _CCH, chunk, 0)

    return _sc_combine


def _sc_gather(flat, src_tok):
    return _make_sc_gather()(flat, src_tok)


def _sc_combine(y, d1, d2):
    return _make_sc_combine()(y, d1, d2)


def kernel(x, switch_w, switch_b, gate_w, up_w, down_w):
    bsz, n, d = x.shape
    flat = x.reshape(-1, d)

    e1, e2, w1, w2 = _run_router(flat, switch_w, switch_b)
    e1 = e1[:, 0]
    e2 = e2[:, 0]

    a = jnp.concatenate([e1, e2])
    oh = jax.nn.one_hot(a, NE, dtype=jnp.int32)
    ranks = jnp.cumsum(oh, axis=0) - oh
    rank = jnp.take_along_axis(ranks, a[:, None], axis=1)[:, 0]
    counts = jnp.sum(oh, axis=0)
    padded = ((counts + TM - 1) // TM) * TM
    cum_end = jnp.cumsum(padded)
    pad_off = cum_end - padded
    dest = pad_off[a] + rank

    present = counts > 0
    nruns = jnp.sum(present.astype(jnp.int32))
    rr = jnp.cumsum(present.astype(jnp.int32)) - present.astype(jnp.int32)
    eids = jnp.arange(NE, dtype=jnp.int32)
    run_e = jnp.zeros((NE,), jnp.int32).at[
        jnp.where(present, rr, NE)].set(eids, mode="drop")
    run_b0 = (pad_off // TM)[run_e].astype(jnp.int32)
    run_nb = jnp.where(eids < nruns, (padded // TM)[run_e], 0).astype(
        jnp.int32)
    meta = jnp.concatenate(
        [nruns[None], run_e, run_b0, run_nb]).astype(jnp.int32)

    tok = jnp.concatenate([jnp.arange(T, dtype=jnp.int32)] * 2)
    src_tok = jnp.zeros((GPAD,), jnp.int32).at[dest].set(tok)
    cw_assign = jnp.concatenate([w1[:, 0], w2[:, 0]])
    cw_col = jnp.zeros((GPAD, 1), jnp.float32).at[dest, 0].set(cw_assign)

    sorted_x = _sc_gather(flat, src_tok)
    y = _run_ffn(sorted_x.astype(jnp.bfloat16), gate_w, up_w, down_w,
                 cw_col, meta)
    out = _sc_combine(y, dest[:T].astype(jnp.int32),
                      dest[T:].astype(jnp.int32))
    return out.reshape(bsz, n, d)


# speedup vs baseline: 1.7958x; 1.1356x over previous
import functools

import jax
import jax.numpy as jnp
from jax import lax
from jax.experimental import pallas as pl
from jax.experimental.pallas import tpu as pltpu
from jax.experimental.pallas import tpu_sc as plsc

T = 2048
D = 1024
DFF = 2816
NE = 8
EPAD = 128

TM = 128
NB = 40
GPAD = NB * TM
TFF = 256
NF = DFF // TFF

NC = 2
NS = 16
NW = NC * NS


def _router_body(x_ref, w_ref, b_ref, e1_ref, e2_ref, w1_ref, w2_ref):
    x = x_ref[...]
    w = w_ref[...]
    logits = lax.dot_general(x, w, (((1,), (1,)), ((), ())),
                             preferred_element_type=jnp.float32)
    logits = logits + b_ref[...]
    eidx = lax.broadcasted_iota(jnp.int32, (T, EPAD), 1)
    m1 = jnp.max(logits, axis=1, keepdims=True)
    e1 = jnp.min(jnp.where(logits >= m1, eidx, EPAD), axis=1, keepdims=True)
    l2 = jnp.where(eidx == e1, -1e30, logits)
    m2 = jnp.max(l2, axis=1, keepdims=True)
    e2 = jnp.min(jnp.where(l2 >= m2, eidx, EPAD), axis=1, keepdims=True)
    e1_ref[...] = e1
    e2_ref[...] = e2
    w1_ref[...] = jax.nn.sigmoid(m1 - m2)
    w2_ref[...] = jax.nn.sigmoid(m2 - m1)


def _run_router(flat, switch_w, switch_b):
    wpad = jnp.zeros((EPAD, D), jnp.float32).at[:NE].set(switch_w)
    bpad = jnp.full((1, EPAD), -1e30, jnp.float32).at[0, :NE].set(switch_b)
    return pl.pallas_call(
        _router_body,
        out_shape=(
            jax.ShapeDtypeStruct((T, 1), jnp.int32),
            jax.ShapeDtypeStruct((T, 1), jnp.int32),
            jax.ShapeDtypeStruct((T, 1), jnp.float32),
            jax.ShapeDtypeStruct((T, 1), jnp.float32),
        ),
    )(flat, wpad, bpad)


_GCH = 32
_GROWS = GPAD // NW


def _sc_mesh():
    return plsc.VectorSubcoreMesh(core_axis_name="c", subcore_axis_name="s",
                                  num_cores=NC, num_subcores=NS)


_AW = (2 * T) // NW
_ACH = 32


@functools.cache
def _make_sc_gather():
    nch = _AW // _ACH

    @functools.partial(
        pl.kernel,
        out_type=jax.ShapeDtypeStruct((GPAD, D), jnp.float32),
        mesh=_sc_mesh(),
        scratch_types=[
            pltpu.VMEM((_ACH,), jnp.int32),
            pltpu.VMEM((_ACH,), jnp.int32),
            pltpu.VMEM((2, _ACH, D), jnp.float32),
            pltpu.SemaphoreType.DMA,
            pltpu.SemaphoreType.DMA,
        ],
    )
    def _sc_gather(x_hbm, dest_hbm, out_hbm, idx0_v, idx1_v, bufs_v, sem0,
                   sem1):
        c = lax.axis_index("c")
        s = lax.axis_index("s")
        w = c * NS + s
        aw = w * _AW
        xrow = lax.rem(aw, T)

        idxs = (idx0_v, idx1_v)
        sems = (sem0, sem1)
        handles = [None] * nch
        for i in range(nch):
            if i >= 2:
                handles[i - 2].wait()
            pltpu.sync_copy(x_hbm.at[pl.ds(xrow + i * _ACH, _ACH)],
                            bufs_v.at[i % 2])
            pltpu.sync_copy(dest_hbm.at[pl.ds(aw + i * _ACH, _ACH)],
                            idxs[i % 2])
            handles[i] = pltpu.async_copy(bufs_v.at[i % 2],
                                          out_hbm.at[idxs[i % 2]],
                                          sems[i % 2])
        handles[nch - 2].wait()
        handles[nch - 1].wait()

    return _sc_gather


def _ffn_body(meta_ref, x_ref, gw_hbm, uw_hbm, dw_hbm, cw_ref, y_ref,
              gb0, gb1, ub0, ub1, db0, db1, g16, u16, d16,
              gs0, gs1, us0, us1, ds0, ds1):
    nruns = meta_ref[0]
    kmax = NF * nruns
    gbufs = (gb0, gb1)
    ubufs = (ub0, ub1)
    dbufs = (db0, db1)
    gsems = (gs0, gs1)
    usems = (us0, us1)
    dsems = (ds0, ds1)

    def issue(slot, e, f):
        foff = f * TFF
        pltpu.async_copy(gw_hbm.at[e, pl.ds(foff, TFF), :], gbufs[slot],
                         gsems[slot])
        pltpu.async_copy(uw_hbm.at[e, pl.ds(foff, TFF), :], ubufs[slot],
                         usems[slot])
        pltpu.async_copy(dw_hbm.at[e, :, pl.ds(foff, TFF)], dbufs[slot],
                         dsems[slot])

    def wait(slot):
        pltpu.make_async_copy(gw_hbm.at[0, pl.ds(0, TFF), :], gbufs[slot],
                              gsems[slot]).wait()
        pltpu.make_async_copy(uw_hbm.at[0, pl.ds(0, TFF), :], ubufs[slot],
                              usems[slot]).wait()
        pltpu.make_async_copy(dw_hbm.at[0, :, pl.ds(0, TFF)], dbufs[slot],
                              dsems[slot]).wait()

    issue(0, meta_ref[1], 0)

    def chunk(k, carry):
        f = k // nruns
        r = k - f * nruns
        kn = k + 1

        @pl.when(kn < kmax)
        def _():
            fn = kn // nruns
            rn = kn - fn * nruns
            en = meta_ref[1 + rn]

            @pl.when(kn % 2 == 0)
            def _():
                issue(0, en, fn)

            @pl.when(kn % 2 == 1)
            def _():
                issue(1, en, fn)

        @pl.when(k % 2 == 0)
        def _():
            wait(0)
            g16[...] = gb0[...].astype(jnp.bfloat16)
            u16[...] = ub0[...].astype(jnp.bfloat16)
            d16[...] = db0[...].astype(jnp.bfloat16)

        @pl.when(k % 2 == 1)
        def _():
            wait(1)
            g16[...] = gb1[...].astype(jnp.bfloat16)
            u16[...] = ub1[...].astype(jnp.bfloat16)
            d16[...] = db1[...].astype(jnp.bfloat16)

        b0 = meta_ref[1 + NE + r]
        nb = meta_ref[1 + 2 * NE + r]

        def block(j, carry2):
            rows = pl.ds((b0 + j) * TM, TM)
            xb = x_ref[rows, :]
            g = lax.dot_general(xb, g16[...], (((1,), (1,)), ((), ())),
                                preferred_element_type=jnp.float32)
            u = lax.dot_general(xb, u16[...], (((1,), (1,)), ((), ())),
                                preferred_element_type=jnp.float32)
            p = (g * jax.nn.sigmoid(g) * u).astype(jnp.bfloat16)
            contrib = lax.dot_general(p, d16[...], (((1,), (1,)), ((), ())),
                                      preferred_element_type=jnp.float32)

            @pl.when(f == 0)
            def _():
                y_ref[rows, :] = contrib

            @pl.when(f != 0)
            def _():
                y_ref[rows, :] += contrib

            @pl.when(f == NF - 1)
            def _():
                y_ref[rows, :] *= cw_ref[rows, :]

            return carry2

        lax.fori_loop(0, nb, block, 0)
        return carry

    lax.fori_loop(0, kmax, chunk, 0)


def _run_ffn(sorted_x16, gate_w, up_w, down_w, cw_col, meta):
    return pl.pallas_call(
        _ffn_body,
        in_specs=[
            pl.BlockSpec(memory_space=pltpu.MemorySpace.SMEM),
            pl.BlockSpec(memory_space=pltpu.MemorySpace.VMEM),
            pl.BlockSpec(memory_space=pl.ANY),
            pl.BlockSpec(memory_space=pl.ANY),
            pl.BlockSpec(memory_space=pl.ANY),
            pl.BlockSpec(memory_space=pltpu.MemorySpace.VMEM),
        ],
        out_specs=pl.BlockSpec(memory_space=pltpu.MemorySpace.VMEM),
        out_shape=jax.ShapeDtypeStruct((GPAD, D), jnp.float32),
        scratch_shapes=[
            pltpu.VMEM((TFF, D), jnp.float32),
            pltpu.VMEM((TFF, D), jnp.float32),
            pltpu.VMEM((TFF, D), jnp.float32),
            pltpu.VMEM((TFF, D), jnp.float32),
            pltpu.VMEM((D, TFF), jnp.float32),
            pltpu.VMEM((D, TFF), jnp.float32),
            pltpu.VMEM((TFF, D), jnp.bfloat16),
            pltpu.VMEM((TFF, D), jnp.bfloat16),
            pltpu.VMEM((D, TFF), jnp.bfloat16),
            pltpu.SemaphoreType.DMA,
            pltpu.SemaphoreType.DMA,
            pltpu.SemaphoreType.DMA,
            pltpu.SemaphoreType.DMA,
            pltpu.SemaphoreType.DMA,
            pltpu.SemaphoreType.DMA,
        ],
    )(meta, sorted_x16, gate_w, up_w, down_w, cw_col)


_CCH = 32
_CTOK = T // NW


@functools.cache
def _make_sc_combine():
    @functools.partial(
        pl.kernel,
        out_type=jax.ShapeDtypeStruct((T, D), jnp.float32),
        mesh=_sc_mesh(),
        scratch_types=[
            pltpu.VMEM((_CCH,), jnp.int32),
            pltpu.VMEM((_CCH, D), jnp.float32),
            pltpu.VMEM((_CCH, D), jnp.float32),
            pltpu.SemaphoreType.DMA,
        ],
    )
    def _sc_combine(y_hbm, d1_hbm, d2_hbm, out_hbm, idx_v, rows_v, acc_v,
                    sem):
        c = lax.axis_index("c")
        s = lax.axis_index("s")
        goff = (c * NS + s) * _CTOK

        def chunk(i, carry):
            og = goff + i * _CCH
            pltpu.sync_copy(d1_hbm.at[pl.ds(og, _CCH)], idx_v)
            pltpu.async_copy(y_hbm.at[idx_v], acc_v, sem).wait()
            pltpu.sync_copy(d2_hbm.at[pl.ds(og, _CCH)], idx_v)
            pltpu.async_copy(y_hbm.at[idx_v], rows_v, sem).wait()

            def row_add(r, carry2):
                for k in range(D // 16):
                    sl = pl.ds(16 * k, 16)
                    acc_v[r, sl] += rows_v[r, sl]
                return carry2

            lax.fori_loop(0, _CCH, row_add, 0)
            pltpu.sync_copy(acc_v, out_hbm.at[pl.ds(og, _CCH)])
            return carry

        lax.fori_loop(0, _CTOK // _CCH, chunk, 0)

    return _sc_combine


def _sc_gather(flat, src_tok):
    return _make_sc_gather()(flat, src_tok)


def _sc_combine(y, d1, d2):
    return _make_sc_combine()(y, d1, d2)


def kernel(x, switch_w, switch_b, gate_w, up_w, down_w):
    bsz, n, d = x.shape
    flat = x.reshape(-1, d)

    e1, e2, w1, w2 = _run_router(flat, switch_w, switch_b)
    e1 = e1[:, 0]
    e2 = e2[:, 0]

    a = jnp.concatenate([e1, e2])
    oh = jax.nn.one_hot(a, NE, dtype=jnp.int32)
    ranks = jnp.cumsum(oh, axis=0) - oh
    rank = jnp.take_along_axis(ranks, a[:, None], axis=1)[:, 0]
    counts = jnp.sum(oh, axis=0)
    padded = ((counts + TM - 1) // TM) * TM
    cum_end = jnp.cumsum(padded)
    pad_off = cum_end - padded
    dest = pad_off[a] + rank

    present = counts > 0
    nruns = jnp.sum(present.astype(jnp.int32))
    rr = jnp.cumsum(present.astype(jnp.int32)) - present.astype(jnp.int32)
    eids = jnp.arange(NE, dtype=jnp.int32)
    run_e = jnp.zeros((NE,), jnp.int32).at[
        jnp.where(present, rr, NE)].set(eids, mode="drop")
    run_b0 = (pad_off // TM)[run_e].astype(jnp.int32)
    run_nb = jnp.where(eids < nruns, (padded // TM)[run_e], 0).astype(
        jnp.int32)
    meta = jnp.concatenate(
        [nruns[None], run_e, run_b0, run_nb]).astype(jnp.int32)

    cw_assign = jnp.concatenate([w1[:, 0], w2[:, 0]])
    cw_col = jnp.zeros((GPAD, 1), jnp.float32).at[dest, 0].set(cw_assign)

    sorted_x = _sc_gather(flat, dest.astype(jnp.int32))
    y = _run_ffn(sorted_x.astype(jnp.bfloat16), gate_w, up_w, down_w,
                 cw_col, meta)
    out = _sc_combine(y, dest[:T].astype(jnp.int32),
                      dest[T:].astype(jnp.int32))
    return out.reshape(bsz, n, d)


# speedup vs baseline: 2.4797x; 1.3808x over previous
import functools

import jax
import jax.numpy as jnp
from jax import lax
from jax.experimental import pallas as pl
from jax.experimental.pallas import tpu as pltpu
from jax.experimental.pallas import tpu_sc as plsc

T = 2048
D = 1024
DFF = 2816
NE = 8
EPAD = 128

TM = 256
NB = (2 * T) // TM + NE
GPAD = NB * TM
TFF = 256
NF = DFF // TFF

NC = 2
NS = 16
NW = NC * NS


def _router_body(x_ref, w_ref, b_ref, e1_ref, e2_ref, w1_ref, w2_ref):
    x = x_ref[...]
    w = w_ref[...]
    logits = lax.dot_general(x, w, (((1,), (1,)), ((), ())),
                             preferred_element_type=jnp.float32)
    logits = logits + b_ref[...]
    eidx = lax.broadcasted_iota(jnp.int32, (T, EPAD), 1)
    m1 = jnp.max(logits, axis=1, keepdims=True)
    e1 = jnp.min(jnp.where(logits >= m1, eidx, EPAD), axis=1, keepdims=True)
    l2 = jnp.where(eidx == e1, -1e30, logits)
    m2 = jnp.max(l2, axis=1, keepdims=True)
    e2 = jnp.min(jnp.where(l2 >= m2, eidx, EPAD), axis=1, keepdims=True)
    e1_ref[...] = e1
    e2_ref[...] = e2
    w1_ref[...] = jax.nn.sigmoid(m1 - m2)
    w2_ref[...] = jax.nn.sigmoid(m2 - m1)


def _run_router(flat, switch_w, switch_b):
    wpad = jnp.zeros((EPAD, D), jnp.float32).at[:NE].set(switch_w)
    bpad = jnp.full((1, EPAD), -1e30, jnp.float32).at[0, :NE].set(switch_b)
    return pl.pallas_call(
        _router_body,
        out_shape=(
            jax.ShapeDtypeStruct((T, 1), jnp.int32),
            jax.ShapeDtypeStruct((T, 1), jnp.int32),
            jax.ShapeDtypeStruct((T, 1), jnp.float32),
            jax.ShapeDtypeStruct((T, 1), jnp.float32),
        ),
    )(flat, wpad, bpad)


_GCH = 32
_GROWS = GPAD // NW


def _sc_mesh():
    return plsc.VectorSubcoreMesh(core_axis_name="c", subcore_axis_name="s",
                                  num_cores=NC, num_subcores=NS)


_AW = (2 * T) // NW
_ACH = 32


@functools.cache
def _make_sc_gather():
    nch = _AW // _ACH

    @functools.partial(
        pl.kernel,
        out_type=jax.ShapeDtypeStruct((GPAD, D), jnp.float32),
        mesh=_sc_mesh(),
        scratch_types=[
            pltpu.VMEM((_ACH,), jnp.int32),
            pltpu.VMEM((_ACH,), jnp.int32),
            pltpu.VMEM((2, _ACH, D), jnp.float32),
            pltpu.SemaphoreType.DMA,
            pltpu.SemaphoreType.DMA,
        ],
    )
    def _sc_gather(x_hbm, dest_hbm, out_hbm, idx0_v, idx1_v, bufs_v, sem0,
                   sem1):
        c = lax.axis_index("c")
        s = lax.axis_index("s")
        w = c * NS + s
        aw = w * _AW
        xrow = lax.rem(aw, T)

        idxs = (idx0_v, idx1_v)
        sems = (sem0, sem1)
        handles = [None] * nch
        for i in range(nch):
            if i >= 2:
                handles[i - 2].wait()
            pltpu.sync_copy(x_hbm.at[pl.ds(xrow + i * _ACH, _ACH)],
                            bufs_v.at[i % 2])
            pltpu.sync_copy(dest_hbm.at[pl.ds(aw + i * _ACH, _ACH)],
                            idxs[i % 2])
            handles[i] = pltpu.async_copy(bufs_v.at[i % 2],
                                          out_hbm.at[idxs[i % 2]],
                                          sems[i % 2])
        handles[nch - 2].wait()
        handles[nch - 1].wait()

    return _sc_gather


def _ffn_body(meta_ref, x_ref, gw_hbm, uw_hbm, dw_hbm, cw_ref, y_ref,
              gb0, gb1, ub0, ub1, db0, db1, g16, u16, d16,
              gs0, gs1, us0, us1, ds0, ds1):
    nruns = meta_ref[0]
    kmax = NF * nruns
    gbufs = (gb0, gb1)
    ubufs = (ub0, ub1)
    dbufs = (db0, db1)
    gsems = (gs0, gs1)
    usems = (us0, us1)
    dsems = (ds0, ds1)

    def issue(slot, e, f):
        foff = f * TFF
        pltpu.async_copy(gw_hbm.at[e, pl.ds(foff, TFF), :], gbufs[slot],
                         gsems[slot])
        pltpu.async_copy(uw_hbm.at[e, pl.ds(foff, TFF), :], ubufs[slot],
                         usems[slot])
        pltpu.async_copy(dw_hbm.at[e, :, pl.ds(foff, TFF)], dbufs[slot],
                         dsems[slot])

    def wait(slot):
        pltpu.make_async_copy(gw_hbm.at[0, pl.ds(0, TFF), :], gbufs[slot],
                              gsems[slot]).wait()
        pltpu.make_async_copy(uw_hbm.at[0, pl.ds(0, TFF), :], ubufs[slot],
                              usems[slot]).wait()
        pltpu.make_async_copy(dw_hbm.at[0, :, pl.ds(0, TFF)], dbufs[slot],
                              dsems[slot]).wait()

    issue(0, meta_ref[1], 0)

    def chunk(k, carry):
        f = k // nruns
        r = k - f * nruns
        kn = k + 1

        @pl.when(kn < kmax)
        def _():
            fn = kn // nruns
            rn = kn - fn * nruns
            en = meta_ref[1 + rn]

            @pl.when(kn % 2 == 0)
            def _():
                issue(0, en, fn)

            @pl.when(kn % 2 == 1)
            def _():
                issue(1, en, fn)

        @pl.when(k % 2 == 0)
        def _():
            wait(0)
            g16[...] = gb0[...].astype(jnp.bfloat16)
            u16[...] = ub0[...].astype(jnp.bfloat16)
            d16[...] = db0[...].astype(jnp.bfloat16)

        @pl.when(k % 2 == 1)
        def _():
            wait(1)
            g16[...] = gb1[...].astype(jnp.bfloat16)
            u16[...] = ub1[...].astype(jnp.bfloat16)
            d16[...] = db1[...].astype(jnp.bfloat16)

        b0 = meta_ref[1 + NE + r]
        nb = meta_ref[1 + 2 * NE + r]

        def block(j, carry2):
            rows = pl.ds((b0 + j) * TM, TM)
            xb = x_ref[rows, :]
            g = lax.dot_general(xb, g16[...], (((1,), (1,)), ((), ())),
                                preferred_element_type=jnp.float32)
            u = lax.dot_general(xb, u16[...], (((1,), (1,)), ((), ())),
                                preferred_element_type=jnp.float32)
            p = (g * jax.nn.sigmoid(g) * u).astype(jnp.bfloat16)
            contrib = lax.dot_general(p, d16[...], (((1,), (1,)), ((), ())),
                                      preferred_element_type=jnp.float32)

            @pl.when(f == 0)
            def _():
                y_ref[rows, :] = contrib

            @pl.when(f != 0)
            def _():
                y_ref[rows, :] += contrib

            @pl.when(f == NF - 1)
            def _():
                y_ref[rows, :] *= cw_ref[rows, :]

            return carry2

        lax.fori_loop(0, nb, block, 0)
        return carry

    lax.fori_loop(0, kmax, chunk, 0)


def _run_ffn(sorted_x16, gate_w, up_w, down_w, cw_col, meta):
    return pl.pallas_call(
        _ffn_body,
        in_specs=[
            pl.BlockSpec(memory_space=pltpu.MemorySpace.SMEM),
            pl.BlockSpec(memory_space=pltpu.MemorySpace.VMEM),
            pl.BlockSpec(memory_space=pl.ANY),
            pl.BlockSpec(memory_space=pl.ANY),
            pl.BlockSpec(memory_space=pl.ANY),
            pl.BlockSpec(memory_space=pltpu.MemorySpace.VMEM),
        ],
        out_specs=pl.BlockSpec(memory_space=pltpu.MemorySpace.VMEM),
        out_shape=jax.ShapeDtypeStruct((GPAD, D), jnp.float32),
        scratch_shapes=[
            pltpu.VMEM((TFF, D), jnp.float32),
            pltpu.VMEM((TFF, D), jnp.float32),
            pltpu.VMEM((TFF, D), jnp.float32),
            pltpu.VMEM((TFF, D), jnp.float32),
            pltpu.VMEM((D, TFF), jnp.float32),
            pltpu.VMEM((D, TFF), jnp.float32),
            pltpu.VMEM((TFF, D), jnp.bfloat16),
            pltpu.VMEM((TFF, D), jnp.bfloat16),
            pltpu.VMEM((D, TFF), jnp.bfloat16),
            pltpu.SemaphoreType.DMA,
            pltpu.SemaphoreType.DMA,
            pltpu.SemaphoreType.DMA,
            pltpu.SemaphoreType.DMA,
            pltpu.SemaphoreType.DMA,
            pltpu.SemaphoreType.DMA,
        ],
    )(meta, sorted_x16, gate_w, up_w, down_w, cw_col)


_CCH = 32
_CTOK = T // NW


@functools.cache
def _make_sc_combine():
    @functools.partial(
        pl.kernel,
        out_type=jax.ShapeDtypeStruct((T, D), jnp.float32),
        mesh=_sc_mesh(),
        scratch_types=[
            pltpu.VMEM((_CCH,), jnp.int32),
            pltpu.VMEM((_CCH, D), jnp.float32),
            pltpu.VMEM((_CCH, D), jnp.float32),
            pltpu.SemaphoreType.DMA,
        ],
    )
    def _sc_combine(y_hbm, d1_hbm, d2_hbm, out_hbm, idx_v, rows_v, acc_v,
                    sem):
        c = lax.axis_index("c")
        s = lax.axis_index("s")
        goff = (c * NS + s) * _CTOK

        def chunk(i, carry):
            og = goff + i * _CCH
            pltpu.sync_copy(d1_hbm.at[pl.ds(og, _CCH)], idx_v)
            pltpu.async_copy(y_hbm.at[idx_v], acc_v, sem).wait()
            pltpu.sync_copy(d2_hbm.at[pl.ds(og, _CCH)], idx_v)
            pltpu.async_copy(y_hbm.at[idx_v], rows_v, sem).wait()

            def row_add(r, carry2):
                for k in range(D // 16):
                    sl = pl.ds(16 * k, 16)
                    acc_v[r, sl] += rows_v[r, sl]
                return carry2

            lax.fori_loop(0, _CCH, row_add, 0)
            pltpu.sync_copy(acc_v, out_hbm.at[pl.ds(og, _CCH)])
            return carry

        lax.fori_loop(0, _CTOK // _CCH, chunk, 0)

    return _sc_combine


def _sc_gather(flat, src_tok):
    return _make_sc_gather()(flat, src_tok)


def _sc_combine(y, d1, d2):
    return _make_sc_combine()(y, d1, d2)


def kernel(x, switch_w, switch_b, gate_w, up_w, down_w):
    bsz, n, d = x.shape
    flat = x.reshape(-1, d)

    e1, e2, w1, w2 = _run_router(flat, switch_w, switch_b)
    e1 = e1[:, 0]
    e2 = e2[:, 0]

    a = jnp.concatenate([e1, e2])
    oh = jax.nn.one_hot(a, NE, dtype=jnp.int32)
    ranks = jnp.cumsum(oh, axis=0) - oh
    rank = jnp.take_along_axis(ranks, a[:, None], axis=1)[:, 0]
    counts = jnp.sum(oh, axis=0)
    padded = ((counts + TM - 1) // TM) * TM
    cum_end = jnp.cumsum(padded)
    pad_off = cum_end - padded
    dest = pad_off[a] + rank

    present = counts > 0
    nruns = jnp.sum(present.astype(jnp.int32))
    rr = jnp.cumsum(present.astype(jnp.int32)) - present.astype(jnp.int32)
    eids = jnp.arange(NE, dtype=jnp.int32)
    run_e = jnp.zeros((NE,), jnp.int32).at[
        jnp.where(present, rr, NE)].set(eids, mode="drop")
    run_b0 = (pad_off // TM)[run_e].astype(jnp.int32)
    run_nb = jnp.where(eids < nruns, (padded // TM)[run_e], 0).astype(
        jnp.int32)
    meta = jnp.concatenate(
        [nruns[None], run_e, run_b0, run_nb]).astype(jnp.int32)

    cw_assign = jnp.concatenate([w1[:, 0], w2[:, 0]])
    cw_col = jnp.zeros((GPAD, 1), jnp.float32).at[dest, 0].set(cw_assign)

    sorted_x = _sc_gather(flat, dest.astype(jnp.int32))
    y = _run_ffn(sorted_x.astype(jnp.bfloat16), gate_w, up_w, down_w,
                 cw_col, meta)
    out = _sc_combine(y, dest[:T].astype(jnp.int32),
                      dest[T:].astype(jnp.int32))
    return out.reshape(bsz, n, d)
